# baseline (speedup 1.0000x reference)
import jax
import jax.numpy as jnp
from jax.experimental import pallas as pl
from jax.experimental.pallas import tpu as pltpu


def _body(x_ref, f_ref, o_ref):
    order = x_ref.shape[1]
    acc = None
    for a in range(order):
        xa = x_ref[:, a, :]
        fa = f_ref[a]
        ya = jnp.dot(xa, fa, preferred_element_type=jnp.float32)
        acc = ya if acc is None else acc * ya
    o_ref[:, :] = jnp.sum(acc, axis=1, keepdims=True)


def kernel(indices_tensor, factors):
    B, order, M = indices_tensor.shape
    R = factors.shape[-1]
    BS = 512

    out = pl.pallas_call(
        _body,
        grid=(B // BS,),
        in_specs=[
            pl.BlockSpec((BS, order, M), lambda i: (i, 0, 0)),
            pl.BlockSpec((order, M, R), lambda i: (0, 0, 0)),
        ],
        out_specs=pl.BlockSpec((BS, 1), lambda i: (i, 0)),
        out_shape=jax.ShapeDtypeStruct((B, 1), jnp.float32),
        compiler_params=pltpu.CompilerParams(
            dimension_semantics=("arbitrary",),
        ),
    )(indices_tensor, factors)
    return out[:, 0]


# speedup vs baseline: 5.0510x; 5.0510x over previous
import jax
import jax.numpy as jnp
from jax.experimental import pallas as pl
from jax.experimental.pallas import tpu as pltpu


def _body(x_ref, f_ref, o_ref):
    order = x_ref.shape[0]
    acc = None
    for a in range(order):
        ya = jnp.dot(f_ref[a], x_ref[a], preferred_element_type=jnp.float32)
        acc = ya if acc is None else acc * ya
    o_ref[0, :] = jnp.sum(acc, axis=0)


def kernel(indices_tensor, factors):
    B, order, M = indices_tensor.shape
    R = factors.shape[-1]
    BS = 1024

    xt = jnp.transpose(indices_tensor, (1, 2, 0))
    ft = jnp.transpose(factors, (0, 2, 1))

    out = pl.pallas_call(
        _body,
        grid=(B // BS,),
        in_specs=[
            pl.BlockSpec((order, M, BS), lambda i: (0, 0, i)),
            pl.BlockSpec((order, R, M), lambda i: (0, 0, 0)),
        ],
        out_specs=pl.BlockSpec((1, BS), lambda i: (0, i)),
        out_shape=jax.ShapeDtypeStruct((1, B), jnp.float32),
        compiler_params=pltpu.CompilerParams(
            dimension_semantics=("arbitrary",),
        ),
    )(xt, ft)
    return out[0]
